# Initial kernel scaffold; baseline (speedup 1.0000x reference)
import jax
import jax.numpy as jnp
from jax.experimental import pallas as pl


def kernel(X_in, ones):
    raise NotImplementedError("write your pallas kernel here")


# speedup vs baseline: 1.0972x; 1.0972x over previous
import functools

import jax
import jax.numpy as jnp
from jax import lax
from jax.experimental import pallas as pl
from jax.experimental.pallas import tpu as pltpu
from jax.experimental.pallas import tpu_sc as plsc

DEPTH = 1000
BATCH = 16384

_info = plsc.get_sparse_core_info()
_NC, _NS, _L = _info.num_cores, _info.num_subcores, _info.num_lanes
_NW = _NC * _NS
_ROWS_PER_W = BATCH // _NW
_CHUNK_ROWS = 32
_CHUNK_WORDS = _CHUNK_ROWS * DEPTH
_N_CHUNKS = _ROWS_PER_W // _CHUNK_ROWS
_GROUPS = _CHUNK_ROWS // _L


@functools.partial(
    pl.kernel,
    out_type=jax.ShapeDtypeStruct((BATCH * DEPTH,), jnp.float32),
    mesh=plsc.VectorSubcoreMesh(core_axis_name="c", subcore_axis_name="s"),
    compiler_params=pltpu.CompilerParams(needs_layout_passes=False),
    scratch_types=[
        pltpu.VMEM((_ROWS_PER_W,), jnp.int32),
        pltpu.VMEM((_CHUNK_WORDS,), jnp.float32),
        pltpu.VMEM((_CHUNK_WORDS,), jnp.float32),
        pltpu.SemaphoreType.DMA,
        pltpu.SemaphoreType.DMA,
    ],
)
def _sc_onehot(idx_hbm, out_hbm, idx_v, buf0, buf1, sem0, sem1):
    wid = lax.axis_index("s") * _NC + lax.axis_index("c")
    row0 = wid * _ROWS_PER_W

    pltpu.sync_copy(idx_hbm.at[pl.ds(row0 * 1, _ROWS_PER_W)], idx_v)

    zero16 = jnp.zeros((_L,), jnp.float32)
    one16 = jnp.full((_L,), 1.0, jnp.float32)
    lanes = lax.iota(jnp.int32, _L)

    def zbody(i, carry):
        base = i * (_L * 16)
        for k in range(16):
            buf0[pl.ds(base + k * _L, _L)] = zero16
            buf1[pl.ds(base + k * _L, _L)] = zero16
        return carry

    lax.fori_loop(0, _CHUNK_WORDS // (_L * 16), zbody, 0)

    def chunk_offs(c, g):
        idxv = idx_v[pl.ds(c * _CHUNK_ROWS + g * _L, _L)]
        return (lanes + g * _L) * DEPTH + idxv

    def out_slice(c):
        return out_hbm.at[pl.ds((row0 + c * _CHUNK_ROWS) * DEPTH, _CHUNK_WORDS)]

    bufs = (buf0, buf1)
    sems = (sem0, sem1)

    for b in range(2):
        for g in range(_GROUPS):
            plsc.store_scatter(bufs[b], [chunk_offs(b, g)], one16)
        pltpu.async_copy(bufs[b], out_slice(b), sems[b])

    def lbody(p, carry):
        for b in range(2):
            c = 2 * p + 2 + b
            pltpu.make_async_copy(bufs[b], out_slice(c - 2), sems[b]).wait()
            for g in range(_GROUPS):
                plsc.store_scatter(bufs[b], [chunk_offs(c - 2, g)], zero16)
            for g in range(_GROUPS):
                plsc.store_scatter(bufs[b], [chunk_offs(c, g)], one16)
            pltpu.async_copy(bufs[b], out_slice(c), sems[b])
        return carry

    lax.fori_loop(0, (_N_CHUNKS - 2) // 2, lbody, 0)

    pltpu.make_async_copy(buf0, out_slice(_N_CHUNKS - 2), sem0).wait()
    pltpu.make_async_copy(buf1, out_slice(_N_CHUNKS - 1), sem1).wait()


@jax.jit
def kernel(X_in, ones):
    del ones
    out_flat = _sc_onehot(X_in.astype(jnp.int32))
    return out_flat.reshape(BATCH, DEPTH)


# speedup vs baseline: 1.7802x; 1.6225x over previous
import functools

import jax
import jax.numpy as jnp
from jax import lax
from jax.experimental import pallas as pl
from jax.experimental.pallas import tpu as pltpu
from jax.experimental.pallas import tpu_sc as plsc

DEPTH = 1000
BATCH = 16384

_info = plsc.get_sparse_core_info()
_NC, _NS, _L = _info.num_cores, _info.num_subcores, _info.num_lanes
_NW = _NC * _NS
_ROWS_PER_W = BATCH // _NW
_CHUNK_ROWS = 32
_N_CHUNKS = _ROWS_PER_W // _CHUNK_ROWS
_GROUPS = _CHUNK_ROWS // _L
_FULL_COLS = DEPTH // _L


@functools.partial(
    pl.kernel,
    out_type=jax.ShapeDtypeStruct((BATCH, DEPTH), jnp.float32),
    mesh=plsc.VectorSubcoreMesh(core_axis_name="c", subcore_axis_name="s"),
    compiler_params=pltpu.CompilerParams(needs_layout_passes=False),
    scratch_types=[
        pltpu.VMEM((_ROWS_PER_W,), jnp.int32),
        pltpu.VMEM((_CHUNK_ROWS, DEPTH), jnp.float32),
        pltpu.VMEM((_CHUNK_ROWS, DEPTH), jnp.float32),
        pltpu.SemaphoreType.DMA,
        pltpu.SemaphoreType.DMA,
    ],
)
def _sc_onehot(idx_hbm, out_hbm, idx_v, buf0, buf1, sem0, sem1):
    wid = lax.axis_index("s") * _NC + lax.axis_index("c")
    row0 = wid * _ROWS_PER_W

    pltpu.sync_copy(idx_hbm.at[pl.ds(row0, _ROWS_PER_W)], idx_v)

    zero16 = jnp.zeros((_L,), jnp.float32)
    one16 = jnp.full((_L,), 1.0, jnp.float32)
    lanes = lax.iota(jnp.int32, _L)

    def zbody(r, carry):
        for k in range(_FULL_COLS):
            buf0[r, pl.ds(k * _L, _L)] = zero16
            buf1[r, pl.ds(k * _L, _L)] = zero16
        buf0[r, pl.ds(DEPTH - _L, _L)] = zero16
        buf1[r, pl.ds(DEPTH - _L, _L)] = zero16
        return carry

    lax.fori_loop(0, _CHUNK_ROWS, zbody, 0)

    def chunk_cols(c, g):
        return idx_v[pl.ds(c * _CHUNK_ROWS + g * _L, _L)]

    def out_slice(c):
        return out_hbm.at[pl.ds(row0 + c * _CHUNK_ROWS, _CHUNK_ROWS)]

    bufs = (buf0, buf1)
    sems = (sem0, sem1)

    for b in range(2):
        for g in range(_GROUPS):
            plsc.store_scatter(bufs[b], [g * _L + lanes, chunk_cols(b, g)], one16)
        pltpu.async_copy(bufs[b], out_slice(b), sems[b])

    def lbody(p, carry):
        for b in range(2):
            c = 2 * p + 2 + b
            pltpu.make_async_copy(bufs[b], out_slice(c - 2), sems[b]).wait()
            for g in range(_GROUPS):
                plsc.store_scatter(
                    bufs[b], [g * _L + lanes, chunk_cols(c - 2, g)], zero16)
            for g in range(_GROUPS):
                plsc.store_scatter(
                    bufs[b], [g * _L + lanes, chunk_cols(c, g)], one16)
            pltpu.async_copy(bufs[b], out_slice(c), sems[b])
        return carry

    lax.fori_loop(0, (_N_CHUNKS - 2) // 2, lbody, 0)

    pltpu.make_async_copy(buf0, out_slice(_N_CHUNKS - 2), sem0).wait()
    pltpu.make_async_copy(buf1, out_slice(_N_CHUNKS - 1), sem1).wait()


@jax.jit
def kernel(X_in, ones):
    del ones
    return _sc_onehot(X_in.astype(jnp.int32))
